# baseline (speedup 1.0000x reference)
import functools

import jax
import jax.numpy as jnp
from jax import lax
from jax.experimental import pallas as pl
from jax.experimental.pallas import tpu as pltpu
from jax.experimental.pallas import tpu_sc as plsc

B = 16384
ED = 128
MD = 8
MW = 4 * MD
MTOT = 1040
ELOFF = (0, 16, 72, 240)

_info = plsc.get_sparse_core_info()
NC, NS = _info.num_cores, _info.num_subcores
NW = NC * NS
BPW = B // NW
CH = 128
NCH = BPW // CH
NBUF = 3


def _sc_gather(uid2, mid2, g, a, o, z, uemb, memb, mtab):
    mesh = plsc.VectorSubcoreMesh(core_axis_name="c", subcore_axis_name="s")

    idx2 = lambda: pltpu.VMEM((NCH, CH), jnp.int32)
    idx1 = lambda: pltpu.VMEM((BPW,), jnp.int32)
    rowbuf = lambda: pltpu.VMEM((CH, ED), jnp.float32)

    @functools.partial(
        pl.kernel,
        mesh=mesh,
        compiler_params=pltpu.CompilerParams(needs_layout_passes=False),
        out_type=[
            jax.ShapeDtypeStruct((B, ED), jnp.float32),
            jax.ShapeDtypeStruct((B, ED), jnp.float32),
            jax.ShapeDtypeStruct((MW, B), jnp.float32),
        ],
        scratch_types=(
            [idx2(), idx2()]
            + [idx1() for _ in range(4)]
            + [rowbuf() for _ in range(2 * NBUF)]
            + [pltpu.VMEM((MTOT,), jnp.float32)]
            + [pltpu.VMEM((MW, BPW), jnp.float32)]
            + [pltpu.SemaphoreType.DMA for _ in range(2 * NBUF + 3)]
        ),
    )
    def body(uid2_h, mid2_h, g_h, a_h, o_h, z_h, uemb_h, memb_h, mtab_h,
             ulat_h, mlat_h, meta_h,
             uix, mix, gix, aix, oix, zix,
             ub0, ub1, ub2, mb0, mb1, mb2, mt_v, ms_v,
             s_init, s_meta, st_ms,
             sg_u0, sg_u1, sg_u2, sg_m0, sg_m1, sg_m2):
        ub = (ub0, ub1, ub2)
        mb = (mb0, mb1, mb2)
        sg_u = (sg_u0, sg_u1, sg_u2)
        sg_m = (sg_m0, sg_m1, sg_m2)

        wid = lax.axis_index("s") * NC + lax.axis_index("c")
        base = wid * BPW

        crows = pl.ds(wid * NCH, NCH)
        iu = pltpu.async_copy(uid2_h.at[crows], uix, s_init)
        im = pltpu.async_copy(mid2_h.at[crows], mix, s_init)
        iu.wait()
        im.wait()
        gu = [None] * NBUF
        gm = [None] * NBUF
        for c in range(NBUF):
            gu[c] = pltpu.async_copy(uemb_h.at[uix.at[c]], ub[c], sg_u[c])
            gm[c] = pltpu.async_copy(memb_h.at[mix.at[c]], mb[c], sg_m[c])

        brow = pl.ds(base, BPW)
        metas = [
            pltpu.async_copy(g_h.at[brow], gix, s_meta),
            pltpu.async_copy(a_h.at[brow], aix, s_meta),
            pltpu.async_copy(o_h.at[brow], oix, s_meta),
            pltpu.async_copy(z_h.at[brow], zix, s_meta),
            pltpu.async_copy(mtab_h, mt_v, s_meta),
        ]
        for cp in metas:
            cp.wait()

        def meta_step(sgrp, _):
            pos = pl.ds(sgrp * 16, 16)
            for t, (tix, eoff) in enumerate(zip((gix, aix, oix, zix),
                                                ELOFF)):
                fb = tix[pos] * MD + eoff
                for j in range(MD):
                    ms_v[t * MD + j, pos] = plsc.load_gather(mt_v, [fb + j])
            return ()
        lax.fori_loop(0, BPW // 16, meta_step, ())
        stms = pltpu.async_copy(ms_v, meta_h.at[:, brow], st_ms)

        stu = [None] * NCH
        stm = [None] * NCH
        for c in range(NCH):
            sl = c % NBUF
            rows = pl.ds(base + c * CH, CH)
            gu[sl].wait()
            gm[sl].wait()
            stu[c] = pltpu.async_copy(ub[sl], ulat_h.at[rows], sg_u[sl])
            stm[c] = pltpu.async_copy(mb[sl], mlat_h.at[rows], sg_m[sl])
            nxt = c + NBUF
            if nxt < NCH:
                stu[c].wait()
                stm[c].wait()
                gu[sl] = pltpu.async_copy(uemb_h.at[uix.at[nxt]], ub[sl],
                                          sg_u[sl])
                gm[sl] = pltpu.async_copy(memb_h.at[mix.at[nxt]], mb[sl],
                                          sg_m[sl])
        for c in range(NCH):
            if c + NBUF >= NCH:
                stu[c].wait()
                stm[c].wait()
        stms.wait()

    return body(uid2, mid2, g, a, o, z, uemb, memb, mtab)


BLK = 8192


def _tc_body(u_ref, m_ref, mt_ref, w_ref, b_ref, out_ref):
    t = jnp.dot(u_ref[...], w_ref[0:ED, :],
                preferred_element_type=jnp.float32)
    t += lax.dot_general(mt_ref[...], w_ref[ED:, :],
                         (((0,), (0,)), ((), ())),
                         preferred_element_type=jnp.float32)
    t += b_ref[...][None, :]
    p = t * m_ref[...]
    ones8 = jnp.ones((8, ED), jnp.float32)
    o8 = lax.dot_general(ones8, p, (((1,), (1,)), ((), ())),
                         preferred_element_type=jnp.float32)
    out_ref[...] = o8[0:1, :].reshape(1, 1, BLK)


def _tc_call(ulat, mlat, meta, W, b):
    grid = (B // BLK,)
    row = lambda i: (i, 0)
    return pl.pallas_call(
        _tc_body,
        grid=grid,
        in_specs=[
            pl.BlockSpec((BLK, ED), row),
            pl.BlockSpec((BLK, ED), row),
            pl.BlockSpec((MW, BLK), lambda i: (0, i)),
            pl.BlockSpec((ED + MW, ED), lambda i: (0, 0)),
            pl.BlockSpec((ED,), lambda i: (0,)),
        ],
        out_specs=pl.BlockSpec((1, 1, BLK), lambda i: (i, 0, 0)),
        out_shape=jax.ShapeDtypeStruct((B // BLK, 1, BLK), jnp.float32),
    )(ulat, mlat, meta, W, b).reshape(B)


def kernel(user_id, movie_id, gender, age, occupation, zip_code,
           user_emb, movie_emb, gender_emb, age_emb, occupation_emb, zip_emb,
           W, b):
    mtab = jnp.concatenate(
        [gender_emb.reshape(-1), age_emb.reshape(-1),
         occupation_emb.reshape(-1), zip_emb.reshape(-1)])
    r2 = lambda x: x.reshape(NW * NCH, CH)
    ulat, mlat, meta = _sc_gather(
        r2(user_id), r2(movie_id), gender, age, occupation, zip_code,
        user_emb, movie_emb, mtab)
    return _tc_call(ulat, mlat, meta, W, b)


# speedup vs baseline: 1.0145x; 1.0145x over previous
import functools

import jax
import jax.numpy as jnp
from jax import lax
from jax.experimental import pallas as pl
from jax.experimental.pallas import tpu as pltpu
from jax.experimental.pallas import tpu_sc as plsc

B = 16384
ED = 128
MD = 8
MW = 4 * MD
MTOT = 1040
ELOFF = (0, 16, 72, 240)

_info = plsc.get_sparse_core_info()
NC, NS = _info.num_cores, _info.num_subcores
NW = NC * NS
BPW = B // NW
CH = 128
NCH = BPW // CH
NBUF = 3


def _sc_gather(uid2, mid2, g, a, o, z, uemb, memb, mtab):
    mesh = plsc.VectorSubcoreMesh(core_axis_name="c", subcore_axis_name="s")

    idx2 = lambda: pltpu.VMEM((NCH, CH), jnp.int32)
    idx1 = lambda: pltpu.VMEM((BPW,), jnp.int32)
    rowbuf = lambda: pltpu.VMEM((CH, ED), jnp.float32)

    @functools.partial(
        pl.kernel,
        mesh=mesh,
        compiler_params=pltpu.CompilerParams(needs_layout_passes=False),
        out_type=[
            jax.ShapeDtypeStruct((B, ED), jnp.float32),
            jax.ShapeDtypeStruct((B, ED), jnp.float32),
            jax.ShapeDtypeStruct((MW, B), jnp.float32),
        ],
        scratch_types=(
            [idx2(), idx2()]
            + [idx1() for _ in range(4)]
            + [rowbuf() for _ in range(2 * NBUF)]
            + [pltpu.VMEM((MTOT,), jnp.float32)]
            + [pltpu.VMEM((MW, BPW), jnp.float32)]
            + [pltpu.SemaphoreType.DMA for _ in range(2 * NBUF + 3)]
        ),
    )
    def body(uid2_h, mid2_h, g_h, a_h, o_h, z_h, uemb_h, memb_h, mtab_h,
             ulat_h, mlat_h, meta_h,
             uix, mix, gix, aix, oix, zix,
             ub0, ub1, ub2, mb0, mb1, mb2, mt_v, ms_v,
             s_init, s_meta, st_ms,
             sg_u0, sg_u1, sg_u2, sg_m0, sg_m1, sg_m2):
        ub = (ub0, ub1, ub2)
        mb = (mb0, mb1, mb2)
        sg_u = (sg_u0, sg_u1, sg_u2)
        sg_m = (sg_m0, sg_m1, sg_m2)

        wid = lax.axis_index("s") * NC + lax.axis_index("c")
        base = wid * BPW

        crows = pl.ds(wid * NCH, NCH)
        iu = pltpu.async_copy(uid2_h.at[crows], uix, s_init)
        im = pltpu.async_copy(mid2_h.at[crows], mix, s_init)
        iu.wait()
        im.wait()
        gu = [None] * NBUF
        gm = [None] * NBUF
        for c in range(NBUF):
            gu[c] = pltpu.async_copy(uemb_h.at[uix.at[c]], ub[c], sg_u[c])
            gm[c] = pltpu.async_copy(memb_h.at[mix.at[c]], mb[c], sg_m[c])

        brow = pl.ds(base, BPW)
        metas = [
            pltpu.async_copy(g_h.at[brow], gix, s_meta),
            pltpu.async_copy(a_h.at[brow], aix, s_meta),
            pltpu.async_copy(o_h.at[brow], oix, s_meta),
            pltpu.async_copy(z_h.at[brow], zix, s_meta),
            pltpu.async_copy(mtab_h, mt_v, s_meta),
        ]
        for cp in metas:
            cp.wait()

        def meta_step(sgrp, _):
            pos = pl.ds(sgrp * 16, 16)
            for t, (tix, eoff) in enumerate(zip((gix, aix, oix, zix),
                                                ELOFF)):
                fb = tix[pos] * MD + eoff
                for j in range(MD):
                    ms_v[t * MD + j, pos] = plsc.load_gather(mt_v, [fb + j])
            return ()
        lax.fori_loop(0, BPW // 16, meta_step, ())
        stms = pltpu.async_copy(ms_v, meta_h.at[:, brow], st_ms)

        stu = [None] * NCH
        stm = [None] * NCH
        for c in range(NCH):
            sl = c % NBUF
            rows = pl.ds(base + c * CH, CH)
            gu[sl].wait()
            gm[sl].wait()
            stu[c] = pltpu.async_copy(ub[sl], ulat_h.at[rows], sg_u[sl])
            stm[c] = pltpu.async_copy(mb[sl], mlat_h.at[rows], sg_m[sl])
            nxt = c + NBUF
            if nxt < NCH:
                stu[c].wait()
                stm[c].wait()
                gu[sl] = pltpu.async_copy(uemb_h.at[uix.at[nxt]], ub[sl],
                                          sg_u[sl])
                gm[sl] = pltpu.async_copy(memb_h.at[mix.at[nxt]], mb[sl],
                                          sg_m[sl])
        for c in range(NCH):
            if c + NBUF >= NCH:
                stu[c].wait()
                stm[c].wait()
        stms.wait()

    return body(uid2, mid2, g, a, o, z, uemb, memb, mtab)


BLK = 4096


def _tc_body(u_ref, m_ref, mt_ref, w_ref, b_ref, out_ref):
    t = jnp.dot(u_ref[...], w_ref[0:ED, :],
                preferred_element_type=jnp.float32)
    t += lax.dot_general(mt_ref[...], w_ref[ED:, :],
                         (((0,), (0,)), ((), ())),
                         preferred_element_type=jnp.float32)
    t += b_ref[...][None, :]
    p = t * m_ref[...]
    ones8 = jnp.ones((8, ED), jnp.float32)
    o8 = lax.dot_general(ones8, p, (((1,), (1,)), ((), ())),
                         preferred_element_type=jnp.float32)
    out_ref[...] = o8[0:1, :].reshape(1, 1, BLK)


def _tc_call(ulat, mlat, meta, W, b):
    grid = (B // BLK,)
    row = lambda i: (i, 0)
    return pl.pallas_call(
        _tc_body,
        grid=grid,
        in_specs=[
            pl.BlockSpec((BLK, ED), row),
            pl.BlockSpec((BLK, ED), row),
            pl.BlockSpec((MW, BLK), lambda i: (0, i)),
            pl.BlockSpec((ED + MW, ED), lambda i: (0, 0)),
            pl.BlockSpec((ED,), lambda i: (0,)),
        ],
        out_specs=pl.BlockSpec((1, 1, BLK), lambda i: (i, 0, 0)),
        out_shape=jax.ShapeDtypeStruct((B // BLK, 1, BLK), jnp.float32),
    )(ulat, mlat, meta, W, b).reshape(B)


def kernel(user_id, movie_id, gender, age, occupation, zip_code,
           user_emb, movie_emb, gender_emb, age_emb, occupation_emb, zip_emb,
           W, b):
    mtab = jnp.concatenate(
        [gender_emb.reshape(-1), age_emb.reshape(-1),
         occupation_emb.reshape(-1), zip_emb.reshape(-1)])
    r2 = lambda x: x.reshape(NW * NCH, CH)
    ulat, mlat, meta = _sc_gather(
        r2(user_id), r2(movie_id), gender, age, occupation, zip_code,
        user_emb, movie_emb, mtab)
    return _tc_call(ulat, mlat, meta, W, b)
